# Initial kernel scaffold; baseline (speedup 1.0000x reference)
import jax
import jax.numpy as jnp
from jax.experimental import pallas as pl


def kernel(x, src_idx, dst_idx, path_len, b):
    raise NotImplementedError("write your pallas kernel here")


# speedup vs baseline: 1.4563x; 1.4563x over previous
import functools

import jax
import jax.numpy as jnp
from jax import lax
from jax.experimental import pallas as pl
from jax.experimental.pallas import tpu as pltpu
from jax.experimental.pallas import tpu_sc as plsc

MAXD = 20
L = 16
NCORES = 2
NSUB = 16
NW = NCORES * NSUB
BLK = 6400
ZCH = 16384
CAP = 24576
DCH = 128


@functools.partial(jax.jit, static_argnums=(0, 1, 2, 3, 4))
def _scatter_call(N, Pp, blk, zch, cap, src, dst, pln, bpad):
    rows = -(-N // NW)
    nblk = Pp // blk
    mesh = plsc.VectorSubcoreMesh(
        core_axis_name="c", subcore_axis_name="s",
        num_cores=NCORES, num_subcores=NSUB)

    @functools.partial(
        pl.kernel,
        out_type=jax.ShapeDtypeStruct((N * N,), jnp.float32),
        mesh=mesh,
        compiler_params=pltpu.CompilerParams(needs_layout_passes=False),
        scratch_types=[
            pltpu.VMEM((blk,), jnp.int32),
            pltpu.VMEM((blk,), jnp.int32),
            pltpu.VMEM((blk,), jnp.int32),
            pltpu.VMEM((32,), jnp.float32),
            pltpu.VMEM((zch,), jnp.float32),
            pltpu.VMEM((cap + DCH,), jnp.int32),
            pltpu.VMEM((cap + DCH,), jnp.float32),
            pltpu.VMEM((DCH,), jnp.int32),
            pltpu.VMEM((DCH,), jnp.float32),
            pltpu.SemaphoreType.DMA,
            pltpu.SemaphoreType.DMA,
            pltpu.SemaphoreType.DMA,
        ],
    )
    def k(src_h, dst_h, pln_h, b_h, out_h,
          src_v, dst_v, pln_v, b_v, zero_v, flat_st, val_st,
          idx_b, val_b, zsem, isem, ssem):
        wid = lax.axis_index("s") * NCORES + lax.axis_index("c")
        lo = jnp.minimum(wid * rows, N)
        hi = jnp.minimum(lo + rows, N)
        flo = lo * N
        fhi = hi * N

        z16 = jnp.zeros((L,), jnp.float32)

        def zv_body(i, _):
            zero_v[pl.ds(i * L, L)] = z16
            return 0
        lax.fori_loop(0, zch // L, zv_body, 0)

        nz = (fhi - flo + zch - 1) // zch

        def zi_body(i, _):
            off = jnp.minimum(flo + i * zch, fhi - zch)
            pltpu.make_async_copy(zero_v, out_h.at[pl.ds(off, zch)],
                                  zsem).start()
            return 0
        lax.fori_loop(0, nz, zi_body, 0)

        pltpu.sync_copy(b_h, b_v)

        def zw_body(i, _):
            pltpu.make_async_copy(zero_v, out_h.at[pl.ds(flo, zch)],
                                  zsem).wait()
            return 0
        lax.fori_loop(0, nz, zw_body, 0)

        def fire(off):
            for kk in range(DCH // L):
                idx_b[pl.ds(kk * L, L)] = flat_st[pl.ds(off + kk * L, L)]
                val_b[pl.ds(kk * L, L)] = val_st[pl.ds(off + kk * L, L)]
            cp = pltpu.make_async_copy(val_b, out_h.at[idx_b], ssem)
            cp.start()
            cp.wait()

        def blk_body(ib, cnt):
            base = ib * blk
            c1 = pltpu.make_async_copy(src_h.at[pl.ds(base, blk)], src_v, isem)
            c2 = pltpu.make_async_copy(dst_h.at[pl.ds(base, blk)], dst_v, isem)
            c3 = pltpu.make_async_copy(pln_h.at[pl.ds(base, blk)], pln_v, isem)
            c1.start(); c2.start(); c3.start()
            c1.wait(); c2.wait(); c3.wait()

            def ch_body(j, cnt):
                o = j * L
                s = src_v[pl.ds(o, L)]
                d = dst_v[pl.ds(o, L)]
                p = pln_v[pl.ds(o, L)]
                flat = s * N + d
                m = (flat >= flo) & (flat < fhi)
                bi = jnp.minimum(p, MAXD) - 1
                v = plsc.load_gather(b_v, [bi])
                plsc.store_compressed(flat_st.at[pl.ds(cnt, L)], flat, mask=m)
                plsc.store_compressed(val_st.at[pl.ds(cnt, L)], v, mask=m)
                cnt = cnt + jnp.sum(m.astype(jnp.int32))

                def do_flush(c):
                    nfull = c // DCH

                    def fb(jj, _):
                        fire(jj * DCH)
                        return 0
                    lax.fori_loop(0, nfull, fb, 0)
                    rem = c - nfull * DCH
                    for kk in range(DCH // L):
                        flat_st[pl.ds(kk * L, L)] = (
                            flat_st[pl.ds(nfull * DCH + kk * L, L)])
                        val_st[pl.ds(kk * L, L)] = (
                            val_st[pl.ds(nfull * DCH + kk * L, L)])
                    return rem

                return lax.cond(cnt >= cap - L, do_flush, lambda c: c, cnt)

            return lax.fori_loop(0, blk // L, ch_body, cnt)

        cnt = lax.fori_loop(0, nblk, blk_body, jnp.int32(0))

        @pl.when(cnt > 0)
        def _():
            al = ((cnt - 1) // L) * L
            pos = (cnt - 1) - al
            lsel = lax.iota(jnp.int32, L) == pos
            last_f = jnp.sum(jnp.where(lsel, flat_st[pl.ds(al, L)], 0))
            last_v = jnp.sum(jnp.where(lsel, val_st[pl.ds(al, L)],
                                       jnp.float32(0)))
            ones = lax.iota(jnp.int32, L) >= 0
            for kk in range(DCH // L):
                plsc.store_compressed(
                    flat_st.at[pl.ds(cnt + kk * L, L)],
                    jnp.full((L,), last_f, jnp.int32), mask=ones)
                plsc.store_compressed(
                    val_st.at[pl.ds(cnt + kk * L, L)],
                    jnp.full((L,), last_v, jnp.float32), mask=ones)
            nfull = (cnt + DCH - 1) // DCH

            def fb(jj, _):
                fire(jj * DCH)
                return 0
            lax.fori_loop(0, nfull, fb, 0)

    return k(src, dst, pln, bpad)


def kernel(x, src_idx, dst_idx, path_len, b):
    N = x.shape[0]
    P = src_idx.shape[0]
    src = jnp.asarray(src_idx, jnp.int32)
    dst = jnp.asarray(dst_idx, jnp.int32)
    pln = jnp.asarray(path_len, jnp.int32)
    Pp = -(-P // BLK) * BLK
    if Pp != P:
        pad = Pp - P
        src = jnp.concatenate([src, jnp.full((pad,), N, jnp.int32)])
        dst = jnp.concatenate([dst, jnp.zeros((pad,), jnp.int32)])
        pln = jnp.concatenate([pln, jnp.ones((pad,), jnp.int32)])
    bpad = jnp.zeros((32,), jnp.float32).at[:MAXD].set(
        b.astype(jnp.float32))
    out = _scatter_call(N, Pp, BLK, ZCH, CAP, src, dst, pln, bpad)
    return out.reshape(N, N).astype(x.dtype)


# speedup vs baseline: 1.4685x; 1.0084x over previous
import functools

import jax
import jax.numpy as jnp
from jax import lax
from jax.experimental import pallas as pl
from jax.experimental.pallas import tpu as pltpu
from jax.experimental.pallas import tpu_sc as plsc

MAXD = 20
L = 16
NCORES = 2
NSUB = 16
NW = NCORES * NSUB
BLK = 12800
ZCH = 16384
CAP = 24576
DCH = 128


@functools.partial(jax.jit, static_argnums=(0, 1, 2, 3, 4))
def _scatter_call(N, Pp, blk, zch, cap, src, dst, pln, bpad):
    rows = -(-N // NW)
    nblk = Pp // blk
    mesh = plsc.VectorSubcoreMesh(
        core_axis_name="c", subcore_axis_name="s",
        num_cores=NCORES, num_subcores=NSUB)

    @functools.partial(
        pl.kernel,
        out_type=jax.ShapeDtypeStruct((N * N,), jnp.float32),
        mesh=mesh,
        compiler_params=pltpu.CompilerParams(needs_layout_passes=False),
        scratch_types=[
            pltpu.VMEM((blk,), jnp.int32),
            pltpu.VMEM((blk,), jnp.int32),
            pltpu.VMEM((blk,), jnp.int32),
            pltpu.VMEM((32,), jnp.float32),
            pltpu.VMEM((zch,), jnp.float32),
            pltpu.VMEM((cap + DCH,), jnp.int32),
            pltpu.VMEM((cap + DCH,), jnp.float32),
            pltpu.VMEM((DCH,), jnp.int32),
            pltpu.VMEM((DCH,), jnp.float32),
            pltpu.SemaphoreType.DMA,
            pltpu.SemaphoreType.DMA,
            pltpu.SemaphoreType.DMA,
        ],
    )
    def k(src_h, dst_h, pln_h, b_h, out_h,
          src_v, dst_v, pln_v, b_v, zero_v, flat_st, val_st,
          idx_b, val_b, zsem, isem, ssem):
        wid = lax.axis_index("s") * NCORES + lax.axis_index("c")
        lo = jnp.minimum(wid * rows, N)
        hi = jnp.minimum(lo + rows, N)
        flo = lo * N
        fhi = hi * N

        z16 = jnp.zeros((L,), jnp.float32)

        def zv_body(i, _):
            zero_v[pl.ds(i * L, L)] = z16
            return 0
        lax.fori_loop(0, zch // L, zv_body, 0)

        nz = (fhi - flo + zch - 1) // zch

        def zi_body(i, _):
            off = jnp.minimum(flo + i * zch, fhi - zch)
            pltpu.make_async_copy(zero_v, out_h.at[pl.ds(off, zch)],
                                  zsem).start()
            return 0
        lax.fori_loop(0, nz, zi_body, 0)

        pltpu.sync_copy(b_h, b_v)

        def drain_zeros():
            def zw_body(i, _):
                pltpu.make_async_copy(zero_v, out_h.at[pl.ds(flo, zch)],
                                      zsem).wait()
                return 0
            lax.fori_loop(0, nz, zw_body, 0)

        def fire(off):
            for kk in range(DCH // L):
                idx_b[pl.ds(kk * L, L)] = flat_st[pl.ds(off + kk * L, L)]
                val_b[pl.ds(kk * L, L)] = val_st[pl.ds(off + kk * L, L)]
            cp = pltpu.make_async_copy(val_b, out_h.at[idx_b], ssem)
            cp.start()
            cp.wait()

        def blk_body(ib, carry):
            cnt, drained = carry
            base = ib * blk
            c1 = pltpu.make_async_copy(src_h.at[pl.ds(base, blk)], src_v, isem)
            c2 = pltpu.make_async_copy(dst_h.at[pl.ds(base, blk)], dst_v, isem)
            c3 = pltpu.make_async_copy(pln_h.at[pl.ds(base, blk)], pln_v, isem)
            c1.start(); c2.start(); c3.start()
            c1.wait(); c2.wait(); c3.wait()

            def ch_body(j, carry):
                cnt, drained = carry
                o = j * L
                s = src_v[pl.ds(o, L)]
                d = dst_v[pl.ds(o, L)]
                p = pln_v[pl.ds(o, L)]
                flat = s * N + d
                m = (flat >= flo) & (flat < fhi)
                bi = jnp.minimum(p, MAXD) - 1
                v = plsc.load_gather(b_v, [bi])
                plsc.store_compressed(flat_st.at[pl.ds(cnt, L)], flat, mask=m)
                plsc.store_compressed(val_st.at[pl.ds(cnt, L)], v, mask=m)
                cnt = cnt + jnp.sum(m.astype(jnp.int32))

                def do_flush(carry):
                    c, drained = carry

                    @pl.when(drained == 0)
                    def _():
                        drain_zeros()
                    nfull = c // DCH

                    def fb(jj, _):
                        fire(jj * DCH)
                        return 0
                    lax.fori_loop(0, nfull, fb, 0)
                    rem = c - nfull * DCH
                    for kk in range(DCH // L):
                        flat_st[pl.ds(kk * L, L)] = (
                            flat_st[pl.ds(nfull * DCH + kk * L, L)])
                        val_st[pl.ds(kk * L, L)] = (
                            val_st[pl.ds(nfull * DCH + kk * L, L)])
                    return (rem, jnp.int32(1))

                return lax.cond(cnt >= cap - L, do_flush, lambda c: c,
                                (cnt, drained))

            return lax.fori_loop(0, blk // L, ch_body, (cnt, drained))

        cnt, drained = lax.fori_loop(0, nblk, blk_body,
                                     (jnp.int32(0), jnp.int32(0)))

        @pl.when(drained == 0)
        def _():
            drain_zeros()

        @pl.when(cnt > 0)
        def _():
            al = ((cnt - 1) // L) * L
            pos = (cnt - 1) - al
            lsel = lax.iota(jnp.int32, L) == pos
            last_f = jnp.sum(jnp.where(lsel, flat_st[pl.ds(al, L)], 0))
            last_v = jnp.sum(jnp.where(lsel, val_st[pl.ds(al, L)],
                                       jnp.float32(0)))
            ones = lax.iota(jnp.int32, L) >= 0
            for kk in range(DCH // L):
                plsc.store_compressed(
                    flat_st.at[pl.ds(cnt + kk * L, L)],
                    jnp.full((L,), last_f, jnp.int32), mask=ones)
                plsc.store_compressed(
                    val_st.at[pl.ds(cnt + kk * L, L)],
                    jnp.full((L,), last_v, jnp.float32), mask=ones)
            nfull = (cnt + DCH - 1) // DCH

            def fb(jj, _):
                fire(jj * DCH)
                return 0
            lax.fori_loop(0, nfull, fb, 0)

    return k(src, dst, pln, bpad)


def kernel(x, src_idx, dst_idx, path_len, b):
    N = x.shape[0]
    P = src_idx.shape[0]
    src = jnp.asarray(src_idx, jnp.int32)
    dst = jnp.asarray(dst_idx, jnp.int32)
    pln = jnp.asarray(path_len, jnp.int32)
    Pp = -(-P // BLK) * BLK
    if Pp != P:
        pad = Pp - P
        src = jnp.concatenate([src, jnp.full((pad,), N, jnp.int32)])
        dst = jnp.concatenate([dst, jnp.zeros((pad,), jnp.int32)])
        pln = jnp.concatenate([pln, jnp.ones((pad,), jnp.int32)])
    bpad = jnp.zeros((32,), jnp.float32).at[:MAXD].set(
        b.astype(jnp.float32))
    out = _scatter_call(N, Pp, BLK, ZCH, CAP, src, dst, pln, bpad)
    return out.reshape(N, N).astype(x.dtype)
